# Initial kernel scaffold; baseline (speedup 1.0000x reference)
import jax
import jax.numpy as jnp
from jax.experimental import pallas as pl


def kernel(x, edge_index, W1, b1, W2, b2, W3, b3, M1, mb1, M2, mb2):
    raise NotImplementedError("write your pallas kernel here")


# speedup vs baseline: 13.5468x; 13.5468x over previous
import functools

import jax
import jax.numpy as jnp
from jax import lax
from jax.experimental import pallas as pl
from jax.experimental.pallas import tpu as pltpu
from jax.experimental.pallas import tpu_sc as plsc

N = 10000
D = 128
E = 320000

NC = 2
NS = 16
NW = NC * NS

NP = 10240
RPT = NP // NS

B = 128
NBT = 79
EPW = NBT * B
EP = NW * EPW

_MESH = dict(core_axis_name="c", subcore_axis_name="s", num_cores=NC,
             num_subcores=NS)


def _sc_deg(dst3, z1):
    @functools.partial(
        pl.kernel,
        out_type=jax.ShapeDtypeStruct((NC, NP), jnp.float32),
        mesh=plsc.VectorSubcoreMesh(**_MESH),
        scratch_types=[
            pltpu.VMEM((NBT, B), jnp.int32),
            pltpu.VMEM((B,), jnp.float32),
            pltpu.VMEM_SHARED((NP,), jnp.float32),
        ],
    )
    def k(dst3_hbm, z1_hbm, degp_hbm, dstv, ones_v, dacc):
        c = lax.axis_index("c")
        s = lax.axis_index("s")
        w = s * NC + c
        pltpu.sync_copy(z1_hbm.at[pl.ds(s * RPT, RPT)],
                        dacc.at[pl.ds(s * RPT, RPT)])
        for i in range(B // 16):
            ones_v[pl.ds(i * 16, 16)] = jnp.ones((16,), jnp.float32)
        pltpu.sync_copy(dst3_hbm.at[w], dstv)
        plsc.subcore_barrier()

        def body(j, carry):
            pltpu.sync_copy(ones_v, dacc.at[dstv.at[j]], add=True)
            return carry

        lax.fori_loop(0, NBT, body, 0)
        plsc.subcore_barrier()
        pltpu.sync_copy(dacc.at[pl.ds(s * RPT, RPT)],
                        degp_hbm.at[c, pl.ds(s * RPT, RPT)])

    return k(dst3, z1)


def _make_sc_agg(do_t):
    outs = [jax.ShapeDtypeStruct((NC, NP, D), jnp.float32)]
    scratch = [
        pltpu.VMEM((NBT, B), jnp.int32),
        pltpu.VMEM((NBT, B), jnp.int32),
        pltpu.VMEM((B, D), jnp.float32),
        pltpu.VMEM_SHARED((NP, D), jnp.float32),
        pltpu.SemaphoreType.DMA,
    ]
    if do_t:
        outs.append(jax.ShapeDtypeStruct((NC, NP), jnp.float32))
        scratch += [
            pltpu.VMEM((B,), jnp.float32),
            pltpu.VMEM_SHARED((NP,), jnp.float32),
            pltpu.SemaphoreType.DMA,
        ]

    @functools.partial(
        pl.kernel,
        out_type=tuple(outs) if do_t else outs[0],
        mesh=plsc.VectorSubcoreMesh(**_MESH),
        scratch_types=scratch,
    )
    def k(hs_hbm, src3_hbm, dst3_hbm, dis_hbm, z2_hbm, z1_hbm, *rest):
        if do_t:
            (agg_hbm, t_hbm, srcv, dstv, rows, acc, sem,
             val, tacc, sem2) = rest
        else:
            agg_hbm, srcv, dstv, rows, acc, sem = rest
        c = lax.axis_index("c")
        s = lax.axis_index("s")
        w = s * NC + c
        pltpu.sync_copy(z2_hbm.at[pl.ds(s * RPT, RPT)],
                        acc.at[pl.ds(s * RPT, RPT)])
        if do_t:
            pltpu.sync_copy(z1_hbm.at[pl.ds(s * RPT, RPT)],
                            tacc.at[pl.ds(s * RPT, RPT)])
        pltpu.sync_copy(src3_hbm.at[w], srcv)
        pltpu.sync_copy(dst3_hbm.at[w], dstv)
        plsc.subcore_barrier()

        def body(j, carry):
            pltpu.async_copy(hs_hbm.at[srcv.at[j]], rows, sem).wait()
            pltpu.sync_copy(rows, acc.at[dstv.at[j]], add=True)
            if do_t:
                pltpu.async_copy(dis_hbm.at[dstv.at[j]], val, sem2).wait()
                pltpu.sync_copy(val, tacc.at[srcv.at[j]], add=True)
            return carry

        lax.fori_loop(0, NBT, body, 0)
        plsc.subcore_barrier()
        pltpu.sync_copy(acc.at[pl.ds(s * RPT, RPT)],
                        agg_hbm.at[c, pl.ds(s * RPT, RPT)])
        if do_t:
            pltpu.sync_copy(tacc.at[pl.ds(s * RPT, RPT)],
                            t_hbm.at[c, pl.ds(s * RPT, RPT)])

    return k


_sc_agg_t = _make_sc_agg(True)
_sc_agg = _make_sc_agg(False)


RB = 640
GRID = NP // RB


def _tc1_body(x_ref, w1_ref, degp_ref, hs_ref, dis_ref):
    pid = pl.program_id(0)
    deg = degp_ref[0] + degp_ref[1] + 1.0
    rows = pid * RB + lax.broadcasted_iota(jnp.int32, (RB, 1), 0)
    dis = jnp.where(rows < N, lax.rsqrt(deg), 0.0)
    h = jnp.dot(x_ref[...], w1_ref[...], preferred_element_type=jnp.float32)
    hs_ref[...] = dis * h
    dis_ref[...] = dis


def _tc1(x_pad, w1, degp):
    return pl.pallas_call(
        _tc1_body,
        grid=(GRID,),
        in_specs=[
            pl.BlockSpec((RB, D), lambda i: (i, 0)),
            pl.BlockSpec((D, D), lambda i: (0, 0)),
            pl.BlockSpec((NC, RB, 1), lambda i: (0, i, 0)),
        ],
        out_specs=[
            pl.BlockSpec((RB, D), lambda i: (i, 0)),
            pl.BlockSpec((RB, 1), lambda i: (i, 0)),
        ],
        out_shape=[
            jax.ShapeDtypeStruct((NP, D), jnp.float32),
            jax.ShapeDtypeStruct((NP, 1), jnp.float32),
        ],
    )(x_pad, w1, degp)


def _tc2_body(aggp_ref, hs1_ref, dis_ref, b1_ref, w2_ref, hs2_ref):
    dis = dis_ref[...]
    a = aggp_ref[0] + aggp_ref[1] + hs1_ref[...]
    h1 = jnp.maximum(dis * a + b1_ref[...], 0.0)
    hs2_ref[...] = dis * jnp.dot(h1, w2_ref[...],
                                 preferred_element_type=jnp.float32)


def _tc2(aggp, hs1, dis, b1, w2):
    return pl.pallas_call(
        _tc2_body,
        grid=(GRID,),
        in_specs=[
            pl.BlockSpec((NC, RB, D), lambda i: (0, i, 0)),
            pl.BlockSpec((RB, D), lambda i: (i, 0)),
            pl.BlockSpec((RB, 1), lambda i: (i, 0)),
            pl.BlockSpec((1, D), lambda i: (0, 0)),
            pl.BlockSpec((D, D), lambda i: (0, 0)),
        ],
        out_specs=pl.BlockSpec((RB, D), lambda i: (i, 0)),
        out_shape=jax.ShapeDtypeStruct((NP, D), jnp.float32),
    )(aggp, hs1, dis, b1, w2)


def _tc3_body(aggp_ref, hs2_ref, dis_ref, tp_ref, b2_ref, w3_ref, b3_ref,
              m1_ref, mb1_ref, m2_ref, mb2_ref, out_ref, zacc):
    pid = pl.program_id(0)
    dis = dis_ref[...]
    a = aggp_ref[0] + aggp_ref[1] + hs2_ref[...]
    h2 = jnp.maximum(dis * a + b2_ref[...], 0.0)
    cvec = dis * (tp_ref[0] + tp_ref[1] + dis)
    part = jnp.sum(cvec * h2, axis=0, keepdims=True)

    @pl.when(pid == 0)
    def _():
        zacc[...] = jnp.zeros_like(zacc)

    zacc[...] += part

    @pl.when(pid == GRID - 1)
    def _():
        z = zacc[...] * (1.0 / N)
        g = jnp.dot(z, w3_ref[...], preferred_element_type=jnp.float32)
        g = g + b3_ref[...]
        g = jnp.maximum(
            jnp.dot(g, m1_ref[...], preferred_element_type=jnp.float32)
            + mb1_ref[...], 0.0)
        g = jnp.dot(g, m2_ref[...], preferred_element_type=jnp.float32)
        g = g + mb2_ref[...]
        out_ref[...] = g


def _tc3(aggp, hs2, dis, tp, b2, w3, b3, m1, mb1, m2, mb2):
    vec = pl.BlockSpec((1, D), lambda i: (0, 0))
    mat = pl.BlockSpec((D, D), lambda i: (0, 0))
    return pl.pallas_call(
        _tc3_body,
        grid=(GRID,),
        in_specs=[
            pl.BlockSpec((NC, RB, D), lambda i: (0, i, 0)),
            pl.BlockSpec((RB, D), lambda i: (i, 0)),
            pl.BlockSpec((RB, 1), lambda i: (i, 0)),
            pl.BlockSpec((NC, RB, 1), lambda i: (0, i, 0)),
            vec, mat, vec, mat, vec, mat, vec,
        ],
        out_specs=pl.BlockSpec((1, D), lambda i: (0, 0)),
        out_shape=jax.ShapeDtypeStruct((1, D), jnp.float32),
        scratch_shapes=[pltpu.VMEM((1, D), jnp.float32)],
    )(aggp, hs2, dis, tp, b2, w3, b3, m1, mb1, m2, mb2)


def kernel(x, edge_index, W1, b1, W2, b2, W3, b3, M1, mb1, M2, mb2):
    x_pad = jnp.pad(x, ((0, NP - N), (0, 0)))
    pad = jnp.full((EP - E,), NP - 1, dtype=jnp.int32)
    src3 = jnp.concatenate([edge_index[0], pad]).reshape(NW, NBT, B)
    dst3 = jnp.concatenate([edge_index[1], pad]).reshape(NW, NBT, B)
    z1 = jnp.zeros((NP,), jnp.float32)
    z2 = jnp.zeros((NP, D), jnp.float32)

    degp = _sc_deg(dst3, z1)
    hs1, dis = _tc1(x_pad, W1, degp.reshape(NC, NP, 1))
    agg1, t = _sc_agg_t(hs1, src3, dst3, dis.reshape(NP), z2, z1)
    hs2 = _tc2(agg1, hs1, dis, b1.reshape(1, D), W2)
    agg2 = _sc_agg(hs2, src3, dst3, dis.reshape(NP), z2, z1)
    g = _tc3(agg2, hs2, dis, t.reshape(NC, NP, 1), b2.reshape(1, D),
             W3, b3.reshape(1, D), M1, mb1.reshape(1, D), M2,
             mb2.reshape(1, D))
    return g
